# Initial kernel scaffold; baseline (speedup 1.0000x reference)
import jax
import jax.numpy as jnp
from jax.experimental import pallas as pl


def kernel(atomic_numbers, table):
    raise NotImplementedError("write your pallas kernel here")


# speedup vs baseline: 1.0606x; 1.0606x over previous
import functools

import jax
import jax.numpy as jnp
from jax import lax
from jax.experimental import pallas as pl
from jax.experimental.pallas import tpu as pltpu
from jax.experimental.pallas import tpu_sc as plsc

N_ATOMS = 100000
FEAT = 128
NC, NS = 2, 16
NW = NC * NS
ROWS_W = N_ATOMS // NW
CHUNK = 125
CHUNK_PAD = 128
NCHUNK = ROWS_W // CHUNK

_mesh = plsc.VectorSubcoreMesh(core_axis_name="c", subcore_axis_name="s")


@functools.partial(
    pl.kernel,
    out_type=jax.ShapeDtypeStruct((N_ATOMS, FEAT), jnp.float32),
    mesh=_mesh,
    scratch_types=[
        pltpu.VMEM((NCHUNK, CHUNK_PAD), jnp.int32),
        pltpu.VMEM((CHUNK_PAD, FEAT), jnp.float32),
        pltpu.SemaphoreType.DMA,
    ],
    compiler_params=pltpu.CompilerParams(use_tc_tiling_on_sc=False),
)
def _embed(idx_hbm, table_hbm, out_hbm, idx_v, buf, gsem):
    wid = lax.axis_index("s") * NC + lax.axis_index("c")
    base = wid * ROWS_W
    pltpu.sync_copy(idx_hbm.at[wid], idx_v)

    def do_chunk(j, carry):
        pltpu.async_copy(table_hbm.at[idx_v.at[j]], buf, gsem).wait()
        pltpu.sync_copy(
            buf.at[pl.ds(0, CHUNK)],
            out_hbm.at[pl.ds(base + j * CHUNK, CHUNK)],
        )
        return carry

    lax.fori_loop(0, NCHUNK, do_chunk, 0)


def kernel(atomic_numbers, table):
    idx = atomic_numbers.reshape(NW, NCHUNK, CHUNK).astype(jnp.int32)
    idx = jnp.pad(idx, ((0, 0), (0, 0), (0, CHUNK_PAD - CHUNK)))
    out = _embed(idx, table)
    return out.reshape(N_ATOMS, 1, FEAT)
